# baseline (speedup 1.0000x reference)
import jax
import jax.numpy as jnp
from jax import lax
from jax.experimental import pallas as pl
from jax.experimental.pallas import tpu as pltpu

B = 256
T = 196
D = 768
P = 8192
LP = 5
K = 4
PB = 1024
QB = 8
NEG = float("-inf")


def _copy_keys_body(x_ref, out_ref, xn_ref):
    c = pl.program_id(0)
    blk = x_ref[...]
    out_ref[pl.ds((c + LP) % 8, 1), :, :] = blk
    row = blk[0]

    @pl.when(c == 0)
    def _():
        xn_ref[...] = row

    @pl.when(c > 0)
    def _():
        xn_ref[...] += row

    @pl.when(c == T - 1)
    def _():
        m = xn_ref[...] * jnp.float32(1.0 / T)
        sq = jnp.sum(m * m, axis=1, keepdims=True)
        xn_ref[...] = m * lax.rsqrt(jnp.maximum(sq, 1e-12))


def _copy_and_keys(x_t):
    return pl.pallas_call(
        _copy_keys_body,
        grid=(T,),
        in_specs=[pl.BlockSpec((1, B, D), lambda c: (c, 0, 0))],
        out_specs=[
            pl.BlockSpec((8, B, D), lambda c: ((c + LP) // 8, 0, 0)),
            pl.BlockSpec((B, D), lambda c: (0, 0)),
        ],
        out_shape=[
            jax.ShapeDtypeStruct((T + LP, B, D), jnp.float32),
            jax.ShapeDtypeStruct((B, D), jnp.float32),
        ],
    )(x_t)


def _top4(s, iota, gbase):
    vs, gs = [], []
    for _ in range(K):
        v = jnp.max(s, axis=1, keepdims=True)
        a = jnp.min(jnp.where(s == v, iota, jnp.int32(2 ** 30)),
                    axis=1, keepdims=True)
        vs.append(v)
        gs.append(a + gbase)
        s = jnp.where(iota == a, NEG, s)
    return jnp.concatenate(vs, axis=1), jnp.concatenate(gs, axis=1)


def _sim_topk_body(xn_ref, pk_ref, sim_ref, tv_ref, ti_ref, idx_ref,
                   rv_ref, ri_ref):
    j = pl.program_id(0)
    xn = xn_ref[...]
    pk = pk_ref[...]
    sq = jnp.sum(pk * pk, axis=1, keepdims=True)
    pkn = pk * lax.rsqrt(jnp.maximum(sq, 1e-12))
    s = lax.dot_general(xn, pkn, (((1,), (1,)), ((), ())),
                        preferred_element_type=jnp.float32)
    sim_ref[...] = s

    iota = lax.broadcasted_iota(jnp.int32, (B, PB), 1)
    bv, bi = _top4(s, iota, j * PB)

    @pl.when(j == 0)
    def _():
        rv_ref[...] = bv
        ri_ref[...] = bi

    @pl.when(j > 0)
    def _():
        cv = jnp.concatenate([rv_ref[...], bv], axis=1)
        ci = jnp.concatenate([ri_ref[...], bi], axis=1)
        iota8 = lax.broadcasted_iota(jnp.int32, (B, 2 * K), 1)
        nvs, nis = [], []
        for _ in range(K):
            v = jnp.max(cv, axis=1, keepdims=True)
            a = jnp.min(jnp.where(cv == v, iota8, jnp.int32(2 ** 30)),
                        axis=1, keepdims=True)
            pick = jnp.sum(jnp.where(iota8 == a, ci, 0), axis=1,
                           keepdims=True)
            nvs.append(v)
            nis.append(pick)
            cv = jnp.where(iota8 == a, NEG, cv)
        rv_ref[...] = jnp.concatenate(nvs, axis=1)
        ri_ref[...] = jnp.concatenate(nis, axis=1)

    @pl.when(j == P // PB - 1)
    def _():
        tv_ref[...] = rv_ref[...]
        ti_ref[...] = ri_ref[...]
        idx_ref[...] = ri_ref[:, 0]


def _sim_topk(xn, prompt_key):
    return pl.pallas_call(
        _sim_topk_body,
        grid=(P // PB,),
        in_specs=[
            pl.BlockSpec((B, D), lambda j: (0, 0)),
            pl.BlockSpec((PB, D), lambda j: (j, 0)),
        ],
        out_specs=[
            pl.BlockSpec((B, PB), lambda j: (0, j)),
            pl.BlockSpec((B, K), lambda j: (0, 0)),
            pl.BlockSpec((B, K), lambda j: (0, 0)),
            pl.BlockSpec((B,), lambda j: (0,)),
        ],
        out_shape=[
            jax.ShapeDtypeStruct((B, P), jnp.float32),
            jax.ShapeDtypeStruct((B, K), jnp.float32),
            jax.ShapeDtypeStruct((B, K), jnp.int32),
            jax.ShapeDtypeStruct((B,), jnp.int32),
        ],
        scratch_shapes=[
            pltpu.VMEM((B, K), jnp.float32),
            pltpu.VMEM((B, K), jnp.int32),
        ],
    )(xn, prompt_key)


def _combine_body(idx_ref, *refs):
    b = pl.program_id(0)
    p_refs = refs[:QB]
    out_ref = refs[QB + 1]
    for i in range(QB):
        r = idx_ref[b * QB + i] % QB
        out_ref[0:LP, pl.ds(i, 1), :] = p_refs[i][:, pl.ds(r, 1), :]


def _gather_combine(idx, p_t, out_big):
    grid_spec = pltpu.PrefetchScalarGridSpec(
        num_scalar_prefetch=1,
        grid=(B // QB,),
        in_specs=[
            pl.BlockSpec((LP, QB, D),
                         lambda b, idx_ref, i=i: (0, idx_ref[b * QB + i] // QB, 0))
            for i in range(QB)
        ] + [
            pl.BlockSpec(memory_space=pl.ANY),
        ],
        out_specs=pl.BlockSpec((LP, QB, D), lambda b, idx_ref: (0, b, 0)),
    )
    return pl.pallas_call(
        _combine_body,
        grid_spec=grid_spec,
        out_shape=jax.ShapeDtypeStruct((T + LP, B, D), jnp.float32),
        input_output_aliases={QB + 1: 0},
    )(idx, *([p_t] * QB), out_big)


def kernel(x_embed, prompt, prompt_key):
    x_t = jnp.transpose(x_embed, (1, 0, 2))
    p_t = jnp.transpose(prompt, (1, 0, 2))
    out_big, xn = _copy_and_keys(x_t)
    sim, top_k_sim, top_k_idx, idx = _sim_topk(xn, prompt_key)
    out_t = _gather_combine(idx, p_t, out_big)
    return (sim, top_k_sim, top_k_idx, idx, jnp.transpose(out_t, (1, 0, 2)))


# speedup vs baseline: 1.5470x; 1.5470x over previous
import jax
import jax.numpy as jnp
from jax import lax
from jax.experimental import pallas as pl
from jax.experimental.pallas import tpu as pltpu

B = 256
T = 196
D = 768
P = 8192
LP = 5
K = 4
PB = 1024
QB = 8
NEG = float("-inf")


CS = 14
NC = T // CS
NBUF = 3


def _copy_keys_body(x_hbm, out_hbm, xn_ref, vbuf, in_sems, out_sems):
    c = pl.program_id(0)

    def in_copy(k, slot):
        return pltpu.make_async_copy(
            x_hbm.at[pl.ds(k * CS, CS)], vbuf.at[slot], in_sems.at[slot])

    def out_copy(k, slot):
        return pltpu.make_async_copy(
            vbuf.at[slot], out_hbm.at[pl.ds(k * CS + LP, CS)],
            out_sems.at[slot])

    slot = lax.rem(c, NBUF)
    nslot = lax.rem(c + 1, NBUF)

    @pl.when(c == 0)
    def _():
        in_copy(0, slot).start()

    @pl.when(c >= 2)
    def _():
        out_copy(c - 2, nslot).wait()

    @pl.when(c + 1 < NC)
    def _():
        in_copy(c + 1, nslot).start()

    in_copy(c, slot).wait()
    out_copy(c, slot).start()

    part = jnp.sum(vbuf[slot], axis=0)

    @pl.when(c == 0)
    def _():
        xn_ref[...] = part

    @pl.when(c > 0)
    def _():
        xn_ref[...] += part

    @pl.when(c == NC - 1)
    def _():
        out_copy(NC - 2, lax.rem(NC - 2, NBUF)).wait()
        out_copy(NC - 1, lax.rem(NC - 1, NBUF)).wait()
        m = xn_ref[...] * jnp.float32(1.0 / T)
        sq = jnp.sum(m * m, axis=1, keepdims=True)
        xn_ref[...] = m * lax.rsqrt(jnp.maximum(sq, 1e-12))


def _copy_and_keys(x_t):
    return pl.pallas_call(
        _copy_keys_body,
        grid=(NC,),
        in_specs=[pl.BlockSpec(memory_space=pl.ANY)],
        out_specs=[
            pl.BlockSpec(memory_space=pl.ANY),
            pl.BlockSpec((B, D), lambda c: (0, 0)),
        ],
        out_shape=[
            jax.ShapeDtypeStruct((T + LP, B, D), jnp.float32),
            jax.ShapeDtypeStruct((B, D), jnp.float32),
        ],
        scratch_shapes=[
            pltpu.VMEM((NBUF, CS, B, D), jnp.float32),
            pltpu.SemaphoreType.DMA((NBUF,)),
            pltpu.SemaphoreType.DMA((NBUF,)),
        ],
    )(x_t)


def _top4(s, iota, gbase):
    vs, gs = [], []
    for _ in range(K):
        v = jnp.max(s, axis=1, keepdims=True)
        a = jnp.min(jnp.where(s == v, iota, jnp.int32(2 ** 30)),
                    axis=1, keepdims=True)
        vs.append(v)
        gs.append(a + gbase)
        s = jnp.where(iota == a, NEG, s)
    return jnp.concatenate(vs, axis=1), jnp.concatenate(gs, axis=1)


def _sim_topk_body(xn_ref, pk_ref, sim_ref, tv_ref, ti_ref, idx_ref,
                   rv_ref, ri_ref):
    j = pl.program_id(0)
    xn = xn_ref[...]
    pk = pk_ref[...]
    sq = jnp.sum(pk * pk, axis=1, keepdims=True)
    pkn = pk * lax.rsqrt(jnp.maximum(sq, 1e-12))
    s = lax.dot_general(xn, pkn, (((1,), (1,)), ((), ())),
                        preferred_element_type=jnp.float32)
    sim_ref[...] = s

    iota = lax.broadcasted_iota(jnp.int32, (B, PB), 1)
    bv, bi = _top4(s, iota, j * PB)

    @pl.when(j == 0)
    def _():
        rv_ref[...] = bv
        ri_ref[...] = bi

    @pl.when(j > 0)
    def _():
        cv = jnp.concatenate([rv_ref[...], bv], axis=1)
        ci = jnp.concatenate([ri_ref[...], bi], axis=1)
        iota8 = lax.broadcasted_iota(jnp.int32, (B, 2 * K), 1)
        nvs, nis = [], []
        for _ in range(K):
            v = jnp.max(cv, axis=1, keepdims=True)
            a = jnp.min(jnp.where(cv == v, iota8, jnp.int32(2 ** 30)),
                        axis=1, keepdims=True)
            pick = jnp.sum(jnp.where(iota8 == a, ci, 0), axis=1,
                           keepdims=True)
            nvs.append(v)
            nis.append(pick)
            cv = jnp.where(iota8 == a, NEG, cv)
        rv_ref[...] = jnp.concatenate(nvs, axis=1)
        ri_ref[...] = jnp.concatenate(nis, axis=1)

    @pl.when(j == P // PB - 1)
    def _():
        tv_ref[...] = rv_ref[...]
        ti_ref[...] = ri_ref[...]
        idx_ref[...] = ri_ref[:, 0]


def _sim_topk(xn, prompt_key):
    return pl.pallas_call(
        _sim_topk_body,
        grid=(P // PB,),
        in_specs=[
            pl.BlockSpec((B, D), lambda j: (0, 0)),
            pl.BlockSpec((PB, D), lambda j: (j, 0)),
        ],
        out_specs=[
            pl.BlockSpec((B, PB), lambda j: (0, j)),
            pl.BlockSpec((B, K), lambda j: (0, 0)),
            pl.BlockSpec((B, K), lambda j: (0, 0)),
            pl.BlockSpec((B,), lambda j: (0,)),
        ],
        out_shape=[
            jax.ShapeDtypeStruct((B, P), jnp.float32),
            jax.ShapeDtypeStruct((B, K), jnp.float32),
            jax.ShapeDtypeStruct((B, K), jnp.int32),
            jax.ShapeDtypeStruct((B,), jnp.int32),
        ],
        scratch_shapes=[
            pltpu.VMEM((B, K), jnp.float32),
            pltpu.VMEM((B, K), jnp.int32),
        ],
    )(xn, prompt_key)


def _combine_body(idx_ref, *refs):
    b = pl.program_id(0)
    p_refs = refs[:QB]
    out_ref = refs[QB + 1]
    for i in range(QB):
        r = idx_ref[b * QB + i] % QB
        out_ref[0:LP, pl.ds(i, 1), :] = p_refs[i][:, pl.ds(r, 1), :]


def _gather_combine(idx, p_t, out_big):
    grid_spec = pltpu.PrefetchScalarGridSpec(
        num_scalar_prefetch=1,
        grid=(B // QB,),
        in_specs=[
            pl.BlockSpec((LP, QB, D),
                         lambda b, idx_ref, i=i: (0, idx_ref[b * QB + i] // QB, 0))
            for i in range(QB)
        ] + [
            pl.BlockSpec(memory_space=pl.ANY),
        ],
        out_specs=pl.BlockSpec((LP, QB, D), lambda b, idx_ref: (0, b, 0)),
    )
    return pl.pallas_call(
        _combine_body,
        grid_spec=grid_spec,
        out_shape=jax.ShapeDtypeStruct((T + LP, B, D), jnp.float32),
        input_output_aliases={QB + 1: 0},
    )(idx, *([p_t] * QB), out_big)


def kernel(x_embed, prompt, prompt_key):
    x_t = jnp.transpose(x_embed, (1, 0, 2))
    p_t = jnp.transpose(prompt, (1, 0, 2))
    out_big, xn = _copy_and_keys(x_t)
    sim, top_k_sim, top_k_idx, idx = _sim_topk(xn, prompt_key)
    out_t = _gather_combine(idx, p_t, out_big)
    return (sim, top_k_sim, top_k_idx, idx, jnp.transpose(out_t, (1, 0, 2)))


# speedup vs baseline: 1.6766x; 1.0838x over previous
import jax
import jax.numpy as jnp
from jax import lax
from jax.experimental import pallas as pl
from jax.experimental.pallas import tpu as pltpu

B = 256
T = 196
D = 768
P = 8192
LP = 5
K = 4
PB = 1024
QB = 32
SUBL = 8
NEG = float("-inf")


CS = 14
NC = T // CS
NBUF = 4


def _copy_keys_body(x_hbm, out_hbm, xn_ref, vbuf, in_sems, out_sems):
    c = pl.program_id(0)

    def in_copy(k, slot):
        return pltpu.make_async_copy(
            x_hbm.at[pl.ds(k * CS, CS)], vbuf.at[slot], in_sems.at[slot])

    def out_copy(k, slot):
        return pltpu.make_async_copy(
            vbuf.at[slot], out_hbm.at[pl.ds(k * CS + LP, CS)],
            out_sems.at[slot])

    slot = lax.rem(c, NBUF)
    nslot = lax.rem(c + 1, NBUF)

    @pl.when(c == 0)
    def _():
        in_copy(0, slot).start()

    @pl.when(c >= NBUF - 1)
    def _():
        out_copy(c - (NBUF - 1), nslot).wait()

    @pl.when(c + 1 < NC)
    def _():
        in_copy(c + 1, nslot).start()

    in_copy(c, slot).wait()
    out_copy(c, slot).start()

    part = jnp.sum(vbuf[slot], axis=0)

    @pl.when(c == 0)
    def _():
        xn_ref[...] = part

    @pl.when(c > 0)
    def _():
        xn_ref[...] += part

    @pl.when(c == NC - 1)
    def _():
        for k in range(NC - (NBUF - 1), NC):
            out_copy(k, k % NBUF).wait()
        m = xn_ref[...] * jnp.float32(1.0 / T)
        sq = jnp.sum(m * m, axis=1, keepdims=True)
        xn_ref[...] = m * lax.rsqrt(jnp.maximum(sq, 1e-12))


def _copy_and_keys(x_t):
    return pl.pallas_call(
        _copy_keys_body,
        grid=(NC,),
        in_specs=[pl.BlockSpec(memory_space=pl.ANY)],
        out_specs=[
            pl.BlockSpec(memory_space=pl.ANY),
            pl.BlockSpec((B, D), lambda c: (0, 0)),
        ],
        out_shape=[
            jax.ShapeDtypeStruct((T + LP, B, D), jnp.float32),
            jax.ShapeDtypeStruct((B, D), jnp.float32),
        ],
        scratch_shapes=[
            pltpu.VMEM((NBUF, CS, B, D), jnp.float32),
            pltpu.SemaphoreType.DMA((NBUF,)),
            pltpu.SemaphoreType.DMA((NBUF,)),
        ],
    )(x_t)


def _top4(s, iota, gbase):
    vs, gs = [], []
    for _ in range(K):
        v = jnp.max(s, axis=1, keepdims=True)
        a = jnp.min(jnp.where(s == v, iota, jnp.int32(2 ** 30)),
                    axis=1, keepdims=True)
        vs.append(v)
        gs.append(a + gbase)
        s = jnp.where(iota == a, NEG, s)
    return jnp.concatenate(vs, axis=1), jnp.concatenate(gs, axis=1)


def _sim_topk_body(xn_ref, pk_ref, sim_ref, tv_ref, ti_ref, idx_ref,
                   rv_ref, ri_ref):
    j = pl.program_id(0)
    xn = xn_ref[...]
    pk = pk_ref[...]
    sq = jnp.sum(pk * pk, axis=1, keepdims=True)
    pkn = pk * lax.rsqrt(jnp.maximum(sq, 1e-12))
    s = lax.dot_general(xn, pkn, (((1,), (1,)), ((), ())),
                        preferred_element_type=jnp.float32)
    sim_ref[...] = s

    iota = lax.broadcasted_iota(jnp.int32, (B, PB), 1)
    bv, bi = _top4(s, iota, j * PB)

    @pl.when(j == 0)
    def _():
        rv_ref[...] = bv
        ri_ref[...] = bi

    @pl.when(j > 0)
    def _():
        cv = jnp.concatenate([rv_ref[...], bv], axis=1)
        ci = jnp.concatenate([ri_ref[...], bi], axis=1)
        iota8 = lax.broadcasted_iota(jnp.int32, (B, 2 * K), 1)
        nvs, nis = [], []
        for _ in range(K):
            v = jnp.max(cv, axis=1, keepdims=True)
            a = jnp.min(jnp.where(cv == v, iota8, jnp.int32(2 ** 30)),
                        axis=1, keepdims=True)
            pick = jnp.sum(jnp.where(iota8 == a, ci, 0), axis=1,
                           keepdims=True)
            nvs.append(v)
            nis.append(pick)
            cv = jnp.where(iota8 == a, NEG, cv)
        rv_ref[...] = jnp.concatenate(nvs, axis=1)
        ri_ref[...] = jnp.concatenate(nis, axis=1)

    @pl.when(j == P // PB - 1)
    def _():
        tv_ref[...] = rv_ref[...]
        ti_ref[...] = ri_ref[...]
        idx_ref[...] = ri_ref[:, 0]


def _sim_topk(xn, prompt_key):
    return pl.pallas_call(
        _sim_topk_body,
        grid=(P // PB,),
        in_specs=[
            pl.BlockSpec((B, D), lambda j: (0, 0)),
            pl.BlockSpec((PB, D), lambda j: (j, 0)),
        ],
        out_specs=[
            pl.BlockSpec((B, PB), lambda j: (0, j)),
            pl.BlockSpec((B, K), lambda j: (0, 0)),
            pl.BlockSpec((B, K), lambda j: (0, 0)),
            pl.BlockSpec((B,), lambda j: (0,)),
        ],
        out_shape=[
            jax.ShapeDtypeStruct((B, P), jnp.float32),
            jax.ShapeDtypeStruct((B, K), jnp.float32),
            jax.ShapeDtypeStruct((B, K), jnp.int32),
            jax.ShapeDtypeStruct((B,), jnp.int32),
        ],
        scratch_shapes=[
            pltpu.VMEM((B, K), jnp.float32),
            pltpu.VMEM((B, K), jnp.int32),
        ],
    )(xn, prompt_key)


def _combine_body(idx_ref, *refs):
    b = pl.program_id(0)
    p_refs = refs[:QB]
    out_ref = refs[QB + 1]
    for i in range(QB):
        r = idx_ref[b * QB + i] % SUBL
        out_ref[0:LP, pl.ds(i, 1), :] = p_refs[i][:, pl.ds(r, 1), :]


def _gather_combine(idx, p_t, out_big):
    grid_spec = pltpu.PrefetchScalarGridSpec(
        num_scalar_prefetch=1,
        grid=(B // QB,),
        in_specs=[
            pl.BlockSpec((LP, SUBL, D),
                         lambda b, idx_ref, i=i: (0, idx_ref[b * QB + i] // SUBL, 0))
            for i in range(QB)
        ] + [
            pl.BlockSpec(memory_space=pl.ANY),
        ],
        out_specs=pl.BlockSpec((LP, QB, D), lambda b, idx_ref: (0, b, 0)),
    )
    return pl.pallas_call(
        _combine_body,
        grid_spec=grid_spec,
        out_shape=jax.ShapeDtypeStruct((T + LP, B, D), jnp.float32),
        input_output_aliases={QB + 1: 0},
    )(idx, *([p_t] * QB), out_big)


def kernel(x_embed, prompt, prompt_key):
    x_t = jnp.transpose(x_embed, (1, 0, 2))
    p_t = jnp.transpose(prompt, (1, 0, 2))
    out_big, xn = _copy_and_keys(x_t)
    sim, top_k_sim, top_k_idx, idx = _sim_topk(xn, prompt_key)
    out_t = _gather_combine(idx, p_t, out_big)
    return (sim, top_k_sim, top_k_idx, idx, jnp.transpose(out_t, (1, 0, 2)))


# speedup vs baseline: 1.7600x; 1.0498x over previous
import jax
import jax.numpy as jnp
from jax import lax
from jax.experimental import pallas as pl
from jax.experimental.pallas import tpu as pltpu

B = 256
T = 196
D = 768
P = 8192
LP = 5
K = 4
PB = 2048
QB = 64
SUBL = 8
NEG = float("-inf")


CS = 7
NC = T // CS
NBUF = 8


def _copy_keys_body(x_hbm, out_hbm, xn_ref, vbuf, in_sems, out_sems):
    c = pl.program_id(0)

    def in_copy(k, slot):
        return pltpu.make_async_copy(
            x_hbm.at[pl.ds(k * CS, CS)], vbuf.at[slot], in_sems.at[slot])

    def out_copy(k, slot):
        return pltpu.make_async_copy(
            vbuf.at[slot], out_hbm.at[pl.ds(k * CS + LP, CS)],
            out_sems.at[slot])

    slot = lax.rem(c, NBUF)
    nslot = lax.rem(c + 1, NBUF)

    @pl.when(c == 0)
    def _():
        in_copy(0, slot).start()

    @pl.when(c >= NBUF - 1)
    def _():
        out_copy(c - (NBUF - 1), nslot).wait()

    @pl.when(c + 1 < NC)
    def _():
        in_copy(c + 1, nslot).start()

    in_copy(c, slot).wait()
    out_copy(c, slot).start()

    part = jnp.sum(vbuf[slot], axis=0)

    @pl.when(c == 0)
    def _():
        xn_ref[...] = part

    @pl.when(c > 0)
    def _():
        xn_ref[...] += part

    @pl.when(c == NC - 1)
    def _():
        for k in range(NC - (NBUF - 1), NC):
            out_copy(k, k % NBUF).wait()
        m = xn_ref[...] * jnp.float32(1.0 / T)
        sq = jnp.sum(m * m, axis=1, keepdims=True)
        xn_ref[...] = m * lax.rsqrt(jnp.maximum(sq, 1e-12))


def _copy_and_keys(x_t):
    return pl.pallas_call(
        _copy_keys_body,
        grid=(NC,),
        in_specs=[pl.BlockSpec(memory_space=pl.ANY)],
        out_specs=[
            pl.BlockSpec(memory_space=pl.ANY),
            pl.BlockSpec((B, D), lambda c: (0, 0)),
        ],
        out_shape=[
            jax.ShapeDtypeStruct((T + LP, B, D), jnp.float32),
            jax.ShapeDtypeStruct((B, D), jnp.float32),
        ],
        scratch_shapes=[
            pltpu.VMEM((NBUF, CS, B, D), jnp.float32),
            pltpu.SemaphoreType.DMA((NBUF,)),
            pltpu.SemaphoreType.DMA((NBUF,)),
        ],
    )(x_t)


def _top4(s, iota, gbase):
    vs, gs = [], []
    for _ in range(K):
        v = jnp.max(s, axis=1, keepdims=True)
        a = jnp.min(jnp.where(s == v, iota, jnp.int32(2 ** 30)),
                    axis=1, keepdims=True)
        vs.append(v)
        gs.append(a + gbase)
        s = jnp.where(iota == a, NEG, s)
    return jnp.concatenate(vs, axis=1), jnp.concatenate(gs, axis=1)


def _sim_topk_body(xn_ref, pk_ref, sim_ref, tv_ref, ti_ref, idx_ref,
                   rv_ref, ri_ref):
    j = pl.program_id(0)
    xn = xn_ref[...]
    pk = pk_ref[...]
    sq = jnp.sum(pk * pk, axis=1, keepdims=True)
    pkn = pk * lax.rsqrt(jnp.maximum(sq, 1e-12))
    s = lax.dot_general(xn, pkn, (((1,), (1,)), ((), ())),
                        preferred_element_type=jnp.float32)
    sim_ref[...] = s

    iota = lax.broadcasted_iota(jnp.int32, (B, PB), 1)
    bv, bi = _top4(s, iota, j * PB)

    @pl.when(j == 0)
    def _():
        rv_ref[...] = bv
        ri_ref[...] = bi

    @pl.when(j > 0)
    def _():
        cv = jnp.concatenate([rv_ref[...], bv], axis=1)
        ci = jnp.concatenate([ri_ref[...], bi], axis=1)
        iota8 = lax.broadcasted_iota(jnp.int32, (B, 2 * K), 1)
        nvs, nis = [], []
        for _ in range(K):
            v = jnp.max(cv, axis=1, keepdims=True)
            a = jnp.min(jnp.where(cv == v, iota8, jnp.int32(2 ** 30)),
                        axis=1, keepdims=True)
            pick = jnp.sum(jnp.where(iota8 == a, ci, 0), axis=1,
                           keepdims=True)
            nvs.append(v)
            nis.append(pick)
            cv = jnp.where(iota8 == a, NEG, cv)
        rv_ref[...] = jnp.concatenate(nvs, axis=1)
        ri_ref[...] = jnp.concatenate(nis, axis=1)

    @pl.when(j == P // PB - 1)
    def _():
        tv_ref[...] = rv_ref[...]
        ti_ref[...] = ri_ref[...]
        idx_ref[...] = ri_ref[:, 0]


def _sim_topk(xn, prompt_key):
    return pl.pallas_call(
        _sim_topk_body,
        grid=(P // PB,),
        in_specs=[
            pl.BlockSpec((B, D), lambda j: (0, 0)),
            pl.BlockSpec((PB, D), lambda j: (j, 0)),
        ],
        out_specs=[
            pl.BlockSpec((B, PB), lambda j: (0, j)),
            pl.BlockSpec((B, K), lambda j: (0, 0)),
            pl.BlockSpec((B, K), lambda j: (0, 0)),
            pl.BlockSpec((B,), lambda j: (0,)),
        ],
        out_shape=[
            jax.ShapeDtypeStruct((B, P), jnp.float32),
            jax.ShapeDtypeStruct((B, K), jnp.float32),
            jax.ShapeDtypeStruct((B, K), jnp.int32),
            jax.ShapeDtypeStruct((B,), jnp.int32),
        ],
        scratch_shapes=[
            pltpu.VMEM((B, K), jnp.float32),
            pltpu.VMEM((B, K), jnp.int32),
        ],
    )(xn, prompt_key)


def _combine_body(idx_ref, *refs):
    b = pl.program_id(0)
    p_refs = refs[:QB]
    out_ref = refs[QB + 1]
    for i in range(QB):
        r = idx_ref[b * QB + i] % SUBL
        out_ref[0:LP, pl.ds(i, 1), :] = p_refs[i][:, pl.ds(r, 1), :]


def _gather_combine(idx, p_t, out_big):
    grid_spec = pltpu.PrefetchScalarGridSpec(
        num_scalar_prefetch=1,
        grid=(B // QB,),
        in_specs=[
            pl.BlockSpec((LP, SUBL, D),
                         lambda b, idx_ref, i=i: (0, idx_ref[b * QB + i] // SUBL, 0))
            for i in range(QB)
        ] + [
            pl.BlockSpec(memory_space=pl.ANY),
        ],
        out_specs=pl.BlockSpec((LP, QB, D), lambda b, idx_ref: (0, b, 0)),
    )
    return pl.pallas_call(
        _combine_body,
        grid_spec=grid_spec,
        out_shape=jax.ShapeDtypeStruct((T + LP, B, D), jnp.float32),
        input_output_aliases={QB + 1: 0},
    )(idx, *([p_t] * QB), out_big)


def kernel(x_embed, prompt, prompt_key):
    x_t = jnp.transpose(x_embed, (1, 0, 2))
    p_t = jnp.transpose(prompt, (1, 0, 2))
    out_big, xn = _copy_and_keys(x_t)
    sim, top_k_sim, top_k_idx, idx = _sim_topk(xn, prompt_key)
    out_t = _gather_combine(idx, p_t, out_big)
    return (sim, top_k_sim, top_k_idx, idx, jnp.transpose(out_t, (1, 0, 2)))


# speedup vs baseline: 1.7817x; 1.0123x over previous
import jax
import jax.numpy as jnp
from jax import lax
from jax.experimental import pallas as pl
from jax.experimental.pallas import tpu as pltpu

B = 256
T = 196
D = 768
P = 8192
LP = 5
K = 4
PB = 4096
QB = 64
SUBL = 8
NEG = float("-inf")


CS = 7
NC = T // CS
NBUF = 8


def _copy_keys_body(x_hbm, out_hbm, xn_ref, vbuf, in_sems, out_sems):
    c = pl.program_id(0)

    def in_copy(k, slot):
        return pltpu.make_async_copy(
            x_hbm.at[pl.ds(k * CS, CS)], vbuf.at[slot], in_sems.at[slot])

    def out_copy(k, slot):
        return pltpu.make_async_copy(
            vbuf.at[slot], out_hbm.at[pl.ds(k * CS + LP, CS)],
            out_sems.at[slot])

    slot = lax.rem(c, NBUF)
    nslot = lax.rem(c + 1, NBUF)

    @pl.when(c == 0)
    def _():
        in_copy(0, slot).start()

    @pl.when(c >= NBUF - 1)
    def _():
        out_copy(c - (NBUF - 1), nslot).wait()

    @pl.when(c + 1 < NC)
    def _():
        in_copy(c + 1, nslot).start()

    in_copy(c, slot).wait()
    out_copy(c, slot).start()

    part = jnp.sum(vbuf[slot], axis=0)

    @pl.when(c == 0)
    def _():
        xn_ref[...] = part

    @pl.when(c > 0)
    def _():
        xn_ref[...] += part

    @pl.when(c == NC - 1)
    def _():
        for k in range(NC - (NBUF - 1), NC):
            out_copy(k, k % NBUF).wait()
        m = xn_ref[...] * jnp.float32(1.0 / T)
        sq = jnp.sum(m * m, axis=1, keepdims=True)
        xn_ref[...] = m * lax.rsqrt(jnp.maximum(sq, 1e-12))


def _copy_and_keys(x_t):
    return pl.pallas_call(
        _copy_keys_body,
        grid=(NC,),
        in_specs=[pl.BlockSpec(memory_space=pl.ANY)],
        out_specs=[
            pl.BlockSpec(memory_space=pl.ANY),
            pl.BlockSpec((B, D), lambda c: (0, 0)),
        ],
        out_shape=[
            jax.ShapeDtypeStruct((T + LP, B, D), jnp.float32),
            jax.ShapeDtypeStruct((B, D), jnp.float32),
        ],
        scratch_shapes=[
            pltpu.VMEM((NBUF, CS, B, D), jnp.float32),
            pltpu.SemaphoreType.DMA((NBUF,)),
            pltpu.SemaphoreType.DMA((NBUF,)),
        ],
    )(x_t)


def _top4(s, iota, gbase):
    vs, gs = [], []
    for _ in range(K):
        v = jnp.max(s, axis=1, keepdims=True)
        a = jnp.min(jnp.where(s == v, iota, jnp.int32(2 ** 30)),
                    axis=1, keepdims=True)
        vs.append(v)
        gs.append(a + gbase)
        s = jnp.where(iota == a, NEG, s)
    return jnp.concatenate(vs, axis=1), jnp.concatenate(gs, axis=1)


def _sim_topk_body(xn_ref, pk_ref, sim_ref, tv_ref, ti_ref, idx_ref,
                   rv_ref, ri_ref):
    j = pl.program_id(0)
    xn = xn_ref[...]
    pk = pk_ref[...]
    sq = jnp.sum(pk * pk, axis=1, keepdims=True)
    pkn = pk * lax.rsqrt(jnp.maximum(sq, 1e-12))
    s = lax.dot_general(xn, pkn, (((1,), (1,)), ((), ())),
                        preferred_element_type=jnp.float32)
    sim_ref[...] = s

    iota = lax.broadcasted_iota(jnp.int32, (B, PB), 1)
    bv, bi = _top4(s, iota, j * PB)

    @pl.when(j == 0)
    def _():
        rv_ref[...] = bv
        ri_ref[...] = bi

    @pl.when(j > 0)
    def _():
        cv = jnp.concatenate([rv_ref[...], bv], axis=1)
        ci = jnp.concatenate([ri_ref[...], bi], axis=1)
        iota8 = lax.broadcasted_iota(jnp.int32, (B, 2 * K), 1)
        nvs, nis = [], []
        for _ in range(K):
            v = jnp.max(cv, axis=1, keepdims=True)
            a = jnp.min(jnp.where(cv == v, iota8, jnp.int32(2 ** 30)),
                        axis=1, keepdims=True)
            pick = jnp.sum(jnp.where(iota8 == a, ci, 0), axis=1,
                           keepdims=True)
            nvs.append(v)
            nis.append(pick)
            cv = jnp.where(iota8 == a, NEG, cv)
        rv_ref[...] = jnp.concatenate(nvs, axis=1)
        ri_ref[...] = jnp.concatenate(nis, axis=1)

    @pl.when(j == P // PB - 1)
    def _():
        tv_ref[...] = rv_ref[...]
        ti_ref[...] = ri_ref[...]
        idx_ref[...] = ri_ref[:, 0]


def _sim_topk(xn, prompt_key):
    return pl.pallas_call(
        _sim_topk_body,
        grid=(P // PB,),
        in_specs=[
            pl.BlockSpec((B, D), lambda j: (0, 0)),
            pl.BlockSpec((PB, D), lambda j: (j, 0)),
        ],
        out_specs=[
            pl.BlockSpec((B, PB), lambda j: (0, j)),
            pl.BlockSpec((B, K), lambda j: (0, 0)),
            pl.BlockSpec((B, K), lambda j: (0, 0)),
            pl.BlockSpec((B,), lambda j: (0,)),
        ],
        out_shape=[
            jax.ShapeDtypeStruct((B, P), jnp.float32),
            jax.ShapeDtypeStruct((B, K), jnp.float32),
            jax.ShapeDtypeStruct((B, K), jnp.int32),
            jax.ShapeDtypeStruct((B,), jnp.int32),
        ],
        scratch_shapes=[
            pltpu.VMEM((B, K), jnp.float32),
            pltpu.VMEM((B, K), jnp.int32),
        ],
    )(xn, prompt_key)


def _combine_body(idx_ref, *refs):
    b = pl.program_id(0)
    p_refs = refs[:QB]
    out_ref = refs[QB + 1]
    for i in range(QB):
        r = idx_ref[b * QB + i] % SUBL
        out_ref[0:LP, pl.ds(i, 1), :] = p_refs[i][:, pl.ds(r, 1), :]


def _gather_combine(idx, p_t, out_big):
    grid_spec = pltpu.PrefetchScalarGridSpec(
        num_scalar_prefetch=1,
        grid=(B // QB,),
        in_specs=[
            pl.BlockSpec((LP, SUBL, D),
                         lambda b, idx_ref, i=i: (0, idx_ref[b * QB + i] // SUBL, 0))
            for i in range(QB)
        ] + [
            pl.BlockSpec(memory_space=pl.ANY),
        ],
        out_specs=pl.BlockSpec((LP, QB, D), lambda b, idx_ref: (0, b, 0)),
    )
    return pl.pallas_call(
        _combine_body,
        grid_spec=grid_spec,
        out_shape=jax.ShapeDtypeStruct((T + LP, B, D), jnp.float32),
        input_output_aliases={QB + 1: 0},
    )(idx, *([p_t] * QB), out_big)


def kernel(x_embed, prompt, prompt_key):
    x_t = jnp.transpose(x_embed, (1, 0, 2))
    p_t = jnp.transpose(prompt, (1, 0, 2))
    out_big, xn = _copy_and_keys(x_t)
    sim, top_k_sim, top_k_idx, idx = _sim_topk(xn, prompt_key)
    out_t = _gather_combine(idx, p_t, out_big)
    return (sim, top_k_sim, top_k_idx, idx, jnp.transpose(out_t, (1, 0, 2)))
